# Initial kernel scaffold; baseline (speedup 1.0000x reference)
import jax
import jax.numpy as jnp
from jax.experimental import pallas as pl


def kernel(inputs, pool_idx):
    raise NotImplementedError("write your pallas kernel here")


# speedup vs baseline: 2.3998x; 2.3998x over previous
import functools

import jax
import jax.numpy as jnp
from jax import lax
from jax.experimental import pallas as pl
from jax.experimental.pallas import tpu as pltpu
from jax.experimental.pallas import tpu_sc as plsc

N = 100000
D = 128
C = 40
NCHUNK = N // C
R = 400
NCOPY = N // R
NW = 32


def _pool_body(inputs_hbm, idx_hbm, out_hbm, idx_v, gbuf, cbuf, sem):
    wid = lax.axis_index("s") * 2 + lax.axis_index("c")

    n_copy = jnp.where(wid < NCOPY % NW, NCOPY // NW + 1, NCOPY // NW)

    def copy_body(t, carry):
        c = wid + NW * t
        base = c * R
        pltpu.sync_copy(inputs_hbm.at[pl.ds(base, R)], cbuf)
        pltpu.sync_copy(cbuf, out_hbm.at[pl.ds(base, R)])
        return carry

    lax.fori_loop(0, n_copy, copy_body, 0)

    n_pool = jnp.where(wid < NCHUNK % NW, NCHUNK // NW + 1, NCHUNK // NW)

    def pool_body(t, carry):
        c = wid + NW * t
        base = c * C
        pltpu.sync_copy(idx_hbm.at[c], idx_v)
        pltpu.async_copy(inputs_hbm.at[idx_v], gbuf, sem).wait()

        def row_body(i, rcarry):
            for g in range(D // 16):
                a = gbuf[2 * i, pl.ds(g * 16, 16)]
                b = gbuf[2 * i + 1, pl.ds(g * 16, 16)]
                gbuf[i, pl.ds(g * 16, 16)] = (a + b) * 0.5
            return rcarry

        lax.fori_loop(0, C, row_body, 0)
        pltpu.sync_copy(gbuf.at[pl.ds(0, C)], out_hbm.at[pl.ds(N + base, C)])
        return carry

    lax.fori_loop(0, n_pool, pool_body, 0)


@functools.partial(
    pl.kernel,
    mesh=plsc.VectorSubcoreMesh(core_axis_name="c", subcore_axis_name="s"),
    out_type=jax.ShapeDtypeStruct((2 * N, D), jnp.float32),
    scratch_types=[
        pltpu.VMEM((2 * C,), jnp.int32),
        pltpu.VMEM((2 * C, D), jnp.float32),
        pltpu.VMEM((R, D), jnp.float32),
        pltpu.SemaphoreType.DMA,
    ],
)
def _pooled(inputs_hbm, idx_hbm, out_hbm, idx_v, gbuf, cbuf, sem):
    _pool_body(inputs_hbm, idx_hbm, out_hbm, idx_v, gbuf, cbuf, sem)


def kernel(inputs, pool_idx):
    idx2 = pool_idx.astype(jnp.int32).reshape(NCHUNK, 2 * C)
    return _pooled(inputs, idx2)
